# Initial kernel scaffold; baseline (speedup 1.0000x reference)
import jax
import jax.numpy as jnp
from jax.experimental import pallas as pl


def kernel(users, items, emb_user, emb_item, g_idx, g_val, s_idx, s_val):
    raise NotImplementedError("write your pallas kernel here")


# speedup vs baseline: 2.8880x; 2.8880x over previous
import functools

import jax
import jax.numpy as jnp
from jax import lax
from jax.experimental import pallas as pl
from jax.experimental.pallas import tpu as pltpu
from jax.experimental.pallas import tpu_sc as plsc

_N_USER = 25000
_N = 50000
_D = 64
_H = 32
_E = 800000
_NC = 2
_NS = 16
_NPAD = 51200
_ZROWS = _NPAD // _NS
_SUB = 5
_CW = 128
_EPT = 51200
_EPAD = _EPT * _NS
_EROWS = _EPAD // _CW
_TROWS = _EPT // _CW
_CHUNKS = _TROWS // _SUB
_NQ = 4096
_QPT = _NQ // (_NC * _NS)

_MESH = plsc.VectorSubcoreMesh(core_axis_name="c", subcore_axis_name="s")


def _spmm_body(x_lo, x_hi, row2, col2, val1, zblk, out_lo, out_hi,
               row_v, col_v, val_v, rows_v, acc, sem):
    c = lax.axis_index("c")
    s = lax.axis_index("s")

    pltpu.sync_copy(zblk, acc.at[pl.ds(s * _ZROWS, _ZROWS)])
    plsc.subcore_barrier()

    def run(x_tab):
        base0 = s * _TROWS

        def outer(t, carry):
            base = base0 + t * _SUB
            pltpu.sync_copy(row2.at[pl.ds(base, _SUB)], row_v)
            pltpu.sync_copy(col2.at[pl.ds(base, _SUB)], col_v)
            pltpu.sync_copy(val1.at[pl.ds(base * _CW, _SUB * _CW)], val_v)
            handles = [
                pltpu.async_copy(x_tab.at[col_v.at[j]],
                                 rows_v.at[pl.ds(j * _CW, _CW)], sem)
                for j in range(_SUB)
            ]
            for h in handles:
                h.wait()

            def scale(e, carry2):
                vv = plsc.load_gather(
                    val_v, [jnp.broadcast_to(e, (16,)).astype(jnp.int32)])
                a = rows_v[e, pl.ds(0, 16)]
                b = rows_v[e, pl.ds(16, 16)]
                rows_v[e, pl.ds(0, 16)] = a * vv
                rows_v[e, pl.ds(16, 16)] = b * vv
                return carry2

            lax.fori_loop(0, _SUB * _CW, scale, 0)
            for j in range(_SUB):
                pltpu.sync_copy(rows_v.at[pl.ds(j * _CW, _CW)],
                                acc.at[row_v.at[j]], add=True)
            return carry

        lax.fori_loop(0, _CHUNKS, outer, 0)

    @pl.when(c == 0)
    def _():
        run(x_lo)

    @pl.when(c == 1)
    def _():
        run(x_hi)

    plsc.subcore_barrier()

    @pl.when(c == 0)
    def _():
        pltpu.sync_copy(acc.at[pl.ds(s * _ZROWS, _ZROWS)],
                        out_lo.at[pl.ds(s * _ZROWS, _ZROWS)])

    @pl.when(c == 1)
    def _():
        pltpu.sync_copy(acc.at[pl.ds(s * _ZROWS, _ZROWS)],
                        out_hi.at[pl.ds(s * _ZROWS, _ZROWS)])


_spmm = pl.kernel(
    _spmm_body,
    out_type=(jax.ShapeDtypeStruct((_NPAD, _H), jnp.float32),) * 2,
    mesh=_MESH,
    scratch_types=[
        pltpu.VMEM((_SUB, _CW), jnp.int32),
        pltpu.VMEM((_SUB, _CW), jnp.int32),
        pltpu.VMEM((_SUB * _CW,), jnp.float32),
        pltpu.VMEM((_SUB * _CW, _H), jnp.float32),
        pltpu.VMEM_SHARED((_NPAD, _H), jnp.float32),
        pltpu.SemaphoreType.DMA,
    ],
    compiler_params=pltpu.CompilerParams(needs_layout_passes=False,
                                         use_tc_tiling_on_sc=False),
)


def _att_body(sd_lo, sd_hi, sm_lo, sm_hi, e_lo, e_hi, o_lo, o_hi):
    a, b = sd_lo[...], sd_hi[...]
    p, q = sm_lo[...], sm_hi[...]
    u1 = e_lo[...] + 1.0
    v1 = e_hi[...] + 1.0
    ms = (jnp.sum(a * u1, axis=1) + jnp.sum(b * v1, axis=1)) * (1.0 / _D)
    mm = (jnp.sum(p * u1, axis=1) + jnp.sum(q * v1, axis=1)) * (1.0 / _D)
    es = jnp.exp(ms)
    em = jnp.exp(mm)
    w = (es / (es + em))[:, None]
    o_lo[...] = w * a + (1.0 - w) * p
    o_hi[...] = w * b + (1.0 - w) * q


_BATT = 1024
_attention = pl.pallas_call(
    _att_body,
    grid=(_NPAD // _BATT,),
    in_specs=[pl.BlockSpec((_BATT, _H), lambda i: (i, 0))] * 6,
    out_specs=[pl.BlockSpec((_BATT, _H), lambda i: (i, 0))] * 2,
    out_shape=(jax.ShapeDtypeStruct((_NPAD, _H), jnp.float32),) * 2,
)


def _final_body(x0_lo, x0_hi, e1_lo, e1_hi, sd_lo, sd_hi, sm_lo, sm_hi,
                uu, ii, out,
                uidx, iidx,
                bx0ul, bx0uh, be1ul, be1uh, bsdul, bsduh, bsmul, bsmuh,
                bx0il, bx0ih, be1il, be1ih, bsdil, bsdih, bsmil, bsmih,
                stage, sv, sem):
    c = lax.axis_index("c")
    s = lax.axis_index("s")
    wid = s * _NC + c
    q0 = wid * _QPT
    pltpu.sync_copy(uu.at[pl.ds(q0, _QPT)], uidx)
    pltpu.sync_copy(ii.at[pl.ds(q0, _QPT)], iidx)
    tabs = (x0_lo, x0_hi, e1_lo, e1_hi, sd_lo, sd_hi, sm_lo, sm_hi)
    ubufs = (bx0ul, bx0uh, be1ul, be1uh, bsdul, bsduh, bsmul, bsmuh)
    ibufs = (bx0il, bx0ih, be1il, be1ih, bsdil, bsdih, bsmil, bsmih)
    handles = []
    for tab, bu, bi in zip(tabs, ubufs, ibufs):
        handles.append(pltpu.async_copy(tab.at[uidx], bu, sem))
        handles.append(pltpu.async_copy(tab.at[iidx], bi, sem))
    for h in handles:
        h.wait()

    def light(e, bufs):
        bx0l, bx0h, be1l, be1h, bsdl, bsdh, bsml, bsmh = bufs
        x0g = [bx0l[e, pl.ds(0, 16)], bx0l[e, pl.ds(16, 16)],
               bx0h[e, pl.ds(0, 16)], bx0h[e, pl.ds(16, 16)]]
        e1g = [be1l[e, pl.ds(0, 16)], be1l[e, pl.ds(16, 16)],
               be1h[e, pl.ds(0, 16)], be1h[e, pl.ds(16, 16)]]
        sdg = [bsdl[e, pl.ds(0, 16)], bsdl[e, pl.ds(16, 16)],
               bsdh[e, pl.ds(0, 16)], bsdh[e, pl.ds(16, 16)]]
        smg = [bsml[e, pl.ds(0, 16)], bsml[e, pl.ds(16, 16)],
               bsmh[e, pl.ds(0, 16)], bsmh[e, pl.ds(16, 16)]]
        ts = sdg[0] * (e1g[0] + 1.0)
        tm = smg[0] * (e1g[0] + 1.0)
        for g in range(1, 4):
            ts = ts + sdg[g] * (e1g[g] + 1.0)
            tm = tm + smg[g] * (e1g[g] + 1.0)
        ms = jnp.sum(ts) * (1.0 / _D)
        mm = jnp.sum(tm) * (1.0 / _D)
        es = jnp.exp(jnp.full((16,), ms))
        em = jnp.exp(jnp.full((16,), mm))
        w = es / (es + em)
        one_third = 1.0 / 3.0
        return [(x0g[g] + e1g[g] + w * sdg[g] + (1.0 - w) * smg[g]) * one_third
                for g in range(4)]

    def q(e, carry):
        lu = light(e, ubufs)
        li = light(e, ibufs)
        p = lu[0] * li[0]
        for g in range(1, 4):
            p = p + lu[g] * li[g]
        stage[pl.ds(e * 16, 16)] = p
        return carry

    lax.fori_loop(0, _QPT, q, 0)

    lanes = jnp.arange(16, dtype=jnp.int32) * 16
    for gq in range(_QPT // 16):
        base16 = lanes + (gq * 256)
        acc16 = plsc.load_gather(stage, [base16])
        for d in range(1, 16):
            acc16 = acc16 + plsc.load_gather(stage, [base16 + d])
        sv[pl.ds(gq * 16, 16)] = acc16
    pltpu.sync_copy(sv, out.at[pl.ds(q0, _QPT)])


_final = pl.kernel(
    _final_body,
    out_type=jax.ShapeDtypeStruct((_NQ,), jnp.float32),
    mesh=_MESH,
    scratch_types=[
        pltpu.VMEM((_QPT,), jnp.int32),
        pltpu.VMEM((_QPT,), jnp.int32),
    ] + [pltpu.VMEM((_QPT, _H), jnp.float32)] * 16 + [
        pltpu.VMEM((_QPT * 16,), jnp.float32),
        pltpu.VMEM((_QPT,), jnp.float32),
        pltpu.SemaphoreType.DMA,
    ],
    compiler_params=pltpu.CompilerParams(needs_layout_passes=False,
                                         use_tc_tiling_on_sc=False),
)


def _prep_edges(idx, val):
    pad = _EPAD - _E
    row = jnp.pad(idx[0], (0, pad)).reshape(_EROWS, _CW)
    col = jnp.pad(idx[1], (0, pad)).reshape(_EROWS, _CW)
    v = jnp.pad(val, (0, pad))
    return row, col, v


def kernel(users, items, emb_user, emb_item, g_idx, g_val, s_idx, s_val):
    x0 = jnp.concatenate([emb_user, emb_item], axis=0)
    x0 = jnp.pad(x0, ((0, _NPAD - _N), (0, 0)))
    x_lo = x0[:, :_H]
    x_hi = x0[:, _H:]
    gr, gc, gv = _prep_edges(g_idx, g_val)
    sr, sc, svv = _prep_edges(s_idx, s_val)
    zblk = jnp.zeros((_ZROWS, _H), jnp.float32)

    sd_lo, sd_hi = _spmm(x_lo, x_hi, gr, gc, gv, zblk)
    sm_lo, sm_hi = _spmm(x_lo, x_hi, sr, sc, svv, zblk)
    e1_lo, e1_hi = _attention(sd_lo, sd_hi, sm_lo, sm_hi, x_lo, x_hi)
    sd2_lo, sd2_hi = _spmm(e1_lo, e1_hi, gr, gc, gv, zblk)
    sm2_lo, sm2_hi = _spmm(e1_lo, e1_hi, sr, sc, svv, zblk)

    scores = _final(x_lo, x_hi, e1_lo, e1_hi, sd2_lo, sd2_hi, sm2_lo, sm2_hi,
                    users, items + _N_USER)
    return scores


# speedup vs baseline: 3.8639x; 1.3379x over previous
import functools

import jax
import jax.numpy as jnp
from jax import lax
from jax.experimental import pallas as pl
from jax.experimental.pallas import tpu as pltpu
from jax.experimental.pallas import tpu_sc as plsc

_N_USER = 25000
_N = 50000
_D = 64
_H = 32
_E = 800000
_NC = 2
_NS = 16
_NPAD = 51200
_ZROWS = _NPAD // _NS
_SUB = 2
_CW = 128
_EPT = 52224
_EPAD = _EPT * _NS
_EROWS = _EPAD // _CW
_TROWS = _EPT // _CW
_CHUNKS = _TROWS // _SUB
_NQ = 4096
_QPT = _NQ // (_NC * _NS)

_MESH = plsc.VectorSubcoreMesh(core_axis_name="c", subcore_axis_name="s")


def _spmm_body(x_lo, x_hi, row2, col2, val1, zblk, out_lo, out_hi,
               row0, row1, row2v, col0, col1, col2v, val0, valb1, valb2,
               rb0, rb1, rb2, acc, sg0, sg1, sg2, sw0, sw1, sw2):
    c = lax.axis_index("c")
    s = lax.axis_index("s")

    pltpu.sync_copy(zblk, acc.at[pl.ds(s * _ZROWS, _ZROWS)])
    plsc.subcore_barrier()

    bufs = ((row0, col0, val0, rb0, sg0, sw0),
            (row1, col1, valb1, rb1, sg1, sw1),
            (row2v, col2v, valb2, rb2, sg2, sw2))

    def run(x_tab):
        base0 = s * _TROWS

        def load(t, b):
            row_v, col_v, val_v, rows_v, sg, sw = bufs[b]
            base = base0 + t * _SUB
            pltpu.sync_copy(row2.at[pl.ds(base, _SUB)], row_v)
            pltpu.sync_copy(col2.at[pl.ds(base, _SUB)], col_v)
            pltpu.sync_copy(val1.at[pl.ds(base * _CW, _SUB * _CW)], val_v)

        def fire_gather(b):
            row_v, col_v, val_v, rows_v, sg, sw = bufs[b]
            for j in range(_SUB):
                pltpu.async_copy(x_tab.at[col_v.at[j]],
                                 rows_v.at[pl.ds(j * _CW, _CW)], sg)

        def wait_gather(b):
            row_v, col_v, val_v, rows_v, sg, sw = bufs[b]
            for j in range(_SUB):
                pltpu.make_async_copy(x_tab.at[col_v.at[j]],
                                      rows_v.at[pl.ds(j * _CW, _CW)],
                                      sg).wait()

        def scale(b):
            row_v, col_v, val_v, rows_v, sg, sw = bufs[b]

            @functools.partial(plsc.parallel_loop, 0, _SUB * _CW, unroll=8)
            def _(e):
                vv = plsc.load_gather(
                    val_v, [jnp.broadcast_to(e, (16,)).astype(jnp.int32)])
                a = rows_v[e, pl.ds(0, 16)]
                b2 = rows_v[e, pl.ds(16, 16)]
                rows_v[e, pl.ds(0, 16)] = a * vv
                rows_v[e, pl.ds(16, 16)] = b2 * vv

        def fire_scatter(b):
            row_v, col_v, val_v, rows_v, sg, sw = bufs[b]
            for j in range(_SUB):
                pltpu.async_copy(rows_v.at[pl.ds(j * _CW, _CW)],
                                 acc.at[row_v.at[j]], sw, add=True)

        def wait_scatter(b):
            row_v, col_v, val_v, rows_v, sg, sw = bufs[b]
            for j in range(_SUB):
                pltpu.make_async_copy(rows_v.at[pl.ds(j * _CW, _CW)],
                                      acc.at[row_v.at[j]], sw).wait()

        load(0, 0)
        fire_gather(0)

        def triple(i, carry):
            t0 = i * 3
            for k in range(3):
                t = t0 + k
                bn = (k + 1) % 3

                @pl.when(t >= 2)
                def _(bn=bn):
                    wait_scatter(bn)

                @pl.when(t < _CHUNKS - 1)
                def _(t=t, bn=bn):
                    load(t + 1, bn)
                    fire_gather(bn)

                wait_gather(k)
                scale(k)
                fire_scatter(k)
            return carry

        lax.fori_loop(0, _CHUNKS // 3, triple, 0)
        wait_scatter((_CHUNKS - 2) % 3)
        wait_scatter((_CHUNKS - 1) % 3)

    @pl.when(c == 0)
    def _():
        run(x_lo)

    @pl.when(c == 1)
    def _():
        run(x_hi)

    plsc.subcore_barrier()

    @pl.when(c == 0)
    def _():
        pltpu.sync_copy(acc.at[pl.ds(s * _ZROWS, _ZROWS)],
                        out_lo.at[pl.ds(s * _ZROWS, _ZROWS)])

    @pl.when(c == 1)
    def _():
        pltpu.sync_copy(acc.at[pl.ds(s * _ZROWS, _ZROWS)],
                        out_hi.at[pl.ds(s * _ZROWS, _ZROWS)])


_spmm = pl.kernel(
    _spmm_body,
    out_type=(jax.ShapeDtypeStruct((_NPAD, _H), jnp.float32),) * 2,
    mesh=_MESH,
    scratch_types=(
        [pltpu.VMEM((_SUB, _CW), jnp.int32)] * 3
        + [pltpu.VMEM((_SUB, _CW), jnp.int32)] * 3
        + [pltpu.VMEM((_SUB * _CW,), jnp.float32)] * 3
        + [pltpu.VMEM((_SUB * _CW, _H), jnp.float32)] * 3
        + [pltpu.VMEM_SHARED((_NPAD, _H), jnp.float32)]
        + [pltpu.SemaphoreType.DMA] * 6
    ),
    compiler_params=pltpu.CompilerParams(needs_layout_passes=False,
                                         use_tc_tiling_on_sc=False),
)


def _att_body(sd_lo, sd_hi, sm_lo, sm_hi, e_lo, e_hi, o_lo, o_hi):
    a, b = sd_lo[...], sd_hi[...]
    p, q = sm_lo[...], sm_hi[...]
    u1 = e_lo[...] + 1.0
    v1 = e_hi[...] + 1.0
    ms = (jnp.sum(a * u1, axis=1) + jnp.sum(b * v1, axis=1)) * (1.0 / _D)
    mm = (jnp.sum(p * u1, axis=1) + jnp.sum(q * v1, axis=1)) * (1.0 / _D)
    es = jnp.exp(ms)
    em = jnp.exp(mm)
    w = (es / (es + em))[:, None]
    o_lo[...] = w * a + (1.0 - w) * p
    o_hi[...] = w * b + (1.0 - w) * q


_BATT = 1024
_attention = pl.pallas_call(
    _att_body,
    grid=(_NPAD // _BATT,),
    in_specs=[pl.BlockSpec((_BATT, _H), lambda i: (i, 0))] * 6,
    out_specs=[pl.BlockSpec((_BATT, _H), lambda i: (i, 0))] * 2,
    out_shape=(jax.ShapeDtypeStruct((_NPAD, _H), jnp.float32),) * 2,
)


def _final_body(x0_lo, x0_hi, e1_lo, e1_hi, sd_lo, sd_hi, sm_lo, sm_hi,
                uu, ii, out,
                uidx, iidx,
                bx0ul, bx0uh, be1ul, be1uh, bsdul, bsduh, bsmul, bsmuh,
                bx0il, bx0ih, be1il, be1ih, bsdil, bsdih, bsmil, bsmih,
                stage, sv, sem):
    c = lax.axis_index("c")
    s = lax.axis_index("s")
    wid = s * _NC + c
    q0 = wid * _QPT
    pltpu.sync_copy(uu.at[pl.ds(q0, _QPT)], uidx)
    pltpu.sync_copy(ii.at[pl.ds(q0, _QPT)], iidx)
    tabs = (x0_lo, x0_hi, e1_lo, e1_hi, sd_lo, sd_hi, sm_lo, sm_hi)
    ubufs = (bx0ul, bx0uh, be1ul, be1uh, bsdul, bsduh, bsmul, bsmuh)
    ibufs = (bx0il, bx0ih, be1il, be1ih, bsdil, bsdih, bsmil, bsmih)
    handles = []
    for tab, bu, bi in zip(tabs, ubufs, ibufs):
        handles.append(pltpu.async_copy(tab.at[uidx], bu, sem))
        handles.append(pltpu.async_copy(tab.at[iidx], bi, sem))
    for h in handles:
        h.wait()

    def light(e, bufs):
        bx0l, bx0h, be1l, be1h, bsdl, bsdh, bsml, bsmh = bufs
        x0g = [bx0l[e, pl.ds(0, 16)], bx0l[e, pl.ds(16, 16)],
               bx0h[e, pl.ds(0, 16)], bx0h[e, pl.ds(16, 16)]]
        e1g = [be1l[e, pl.ds(0, 16)], be1l[e, pl.ds(16, 16)],
               be1h[e, pl.ds(0, 16)], be1h[e, pl.ds(16, 16)]]
        sdg = [bsdl[e, pl.ds(0, 16)], bsdl[e, pl.ds(16, 16)],
               bsdh[e, pl.ds(0, 16)], bsdh[e, pl.ds(16, 16)]]
        smg = [bsml[e, pl.ds(0, 16)], bsml[e, pl.ds(16, 16)],
               bsmh[e, pl.ds(0, 16)], bsmh[e, pl.ds(16, 16)]]
        ts = sdg[0] * (e1g[0] + 1.0)
        tm = smg[0] * (e1g[0] + 1.0)
        for g in range(1, 4):
            ts = ts + sdg[g] * (e1g[g] + 1.0)
            tm = tm + smg[g] * (e1g[g] + 1.0)
        ms = jnp.sum(ts) * (1.0 / _D)
        mm = jnp.sum(tm) * (1.0 / _D)
        es = jnp.exp(jnp.full((16,), ms))
        em = jnp.exp(jnp.full((16,), mm))
        w = es / (es + em)
        one_third = 1.0 / 3.0
        return [(x0g[g] + e1g[g] + w * sdg[g] + (1.0 - w) * smg[g]) * one_third
                for g in range(4)]

    def q(e, carry):
        lu = light(e, ubufs)
        li = light(e, ibufs)
        p = lu[0] * li[0]
        for g in range(1, 4):
            p = p + lu[g] * li[g]
        stage[pl.ds(e * 16, 16)] = p
        return carry

    lax.fori_loop(0, _QPT, q, 0)

    lanes = jnp.arange(16, dtype=jnp.int32) * 16
    for gq in range(_QPT // 16):
        base16 = lanes + (gq * 256)
        acc16 = plsc.load_gather(stage, [base16])
        for d in range(1, 16):
            acc16 = acc16 + plsc.load_gather(stage, [base16 + d])
        sv[pl.ds(gq * 16, 16)] = acc16
    pltpu.sync_copy(sv, out.at[pl.ds(q0, _QPT)])


_final = pl.kernel(
    _final_body,
    out_type=jax.ShapeDtypeStruct((_NQ,), jnp.float32),
    mesh=_MESH,
    scratch_types=[
        pltpu.VMEM((_QPT,), jnp.int32),
        pltpu.VMEM((_QPT,), jnp.int32),
    ] + [pltpu.VMEM((_QPT, _H), jnp.float32)] * 16 + [
        pltpu.VMEM((_QPT * 16,), jnp.float32),
        pltpu.VMEM((_QPT,), jnp.float32),
        pltpu.SemaphoreType.DMA,
    ],
    compiler_params=pltpu.CompilerParams(needs_layout_passes=False,
                                         use_tc_tiling_on_sc=False),
)


def _prep_edges(idx, val):
    pad = _EPAD - _E
    row = jnp.pad(idx[0], (0, pad)).reshape(_EROWS, _CW)
    col = jnp.pad(idx[1], (0, pad)).reshape(_EROWS, _CW)
    v = jnp.pad(val, (0, pad))
    return row, col, v


def kernel(users, items, emb_user, emb_item, g_idx, g_val, s_idx, s_val):
    x0 = jnp.concatenate([emb_user, emb_item], axis=0)
    x0 = jnp.pad(x0, ((0, _NPAD - _N), (0, 0)))
    x_lo = x0[:, :_H]
    x_hi = x0[:, _H:]
    gr, gc, gv = _prep_edges(g_idx, g_val)
    sr, sc, svv = _prep_edges(s_idx, s_val)
    zblk = jnp.zeros((_ZROWS, _H), jnp.float32)

    sd_lo, sd_hi = _spmm(x_lo, x_hi, gr, gc, gv, zblk)
    sm_lo, sm_hi = _spmm(x_lo, x_hi, sr, sc, svv, zblk)
    e1_lo, e1_hi = _attention(sd_lo, sd_hi, sm_lo, sm_hi, x_lo, x_hi)
    sd2_lo, sd2_hi = _spmm(e1_lo, e1_hi, gr, gc, gv, zblk)
    sm2_lo, sm2_hi = _spmm(e1_lo, e1_hi, sr, sc, svv, zblk)

    scores = _final(x_lo, x_hi, e1_lo, e1_hi, sd2_lo, sd2_hi, sm2_lo, sm2_hi,
                    users, items + _N_USER)
    return scores


# speedup vs baseline: 4.0320x; 1.0435x over previous
import functools

import jax
import jax.numpy as jnp
from jax import lax
from jax.experimental import pallas as pl
from jax.experimental.pallas import tpu as pltpu
from jax.experimental.pallas import tpu_sc as plsc

_N_USER = 25000
_N = 50000
_D = 64
_H = 32
_E = 800000
_NC = 2
_NS = 16
_NPAD = 51200
_ZROWS = _NPAD // _NS
_SUB = 2
_CW = 128
_EPT = 52224
_EPAD = _EPT * _NS
_EROWS = _EPAD // _CW
_TROWS = _EPT // _CW
_CHUNKS = _TROWS // _SUB
_NQ = 4096
_QPT = _NQ // (_NC * _NS)

_MESH = plsc.VectorSubcoreMesh(core_axis_name="c", subcore_axis_name="s")


def _spmm_body(x_lo, x_hi, ed, zblk, out_lo, out_hi,
               eb0, eb1, eb2, rb0, rb1, rb2, acc,
               sg0, sg1, sg2, sw0, sw1, sw2):
    c = lax.axis_index("c")
    s = lax.axis_index("s")

    pltpu.sync_copy(zblk, acc.at[pl.ds(s * _ZROWS, _ZROWS)])
    plsc.subcore_barrier()

    bufs = ((eb0, rb0, sg0, sw0), (eb1, rb1, sg1, sw1), (eb2, rb2, sg2, sw2))

    def run(x_tab):
        base0 = s * _TROWS

        def load(t, b):
            eb, rows_v, sg, sw = bufs[b]
            base = (base0 + t * _SUB) * 3
            pltpu.sync_copy(ed.at[pl.ds(base, 3 * _SUB)], eb)

        def fire_gather(b):
            eb, rows_v, sg, sw = bufs[b]
            for j in range(_SUB):
                pltpu.async_copy(x_tab.at[eb.at[3 * j + 1]],
                                 rows_v.at[pl.ds(j * _CW, _CW)], sg)

        def wait_gather(b):
            eb, rows_v, sg, sw = bufs[b]
            for j in range(_SUB):
                pltpu.make_async_copy(x_tab.at[eb.at[3 * j + 1]],
                                      rows_v.at[pl.ds(j * _CW, _CW)],
                                      sg).wait()

        def scale(b):
            eb, rows_v, sg, sw = bufs[b]
            for j in range(_SUB):
                jv = jnp.full((16,), 3 * j + 2, jnp.int32)

                @functools.partial(plsc.parallel_loop, 0, _CW, unroll=8)
                def _(k, jv=jv, j=j):
                    kv = jnp.broadcast_to(k, (16,)).astype(jnp.int32)
                    vv = plsc.bitcast(plsc.load_gather(eb, [jv, kv]),
                                      jnp.float32)
                    e = j * _CW + k
                    a = rows_v[e, pl.ds(0, 16)]
                    b2 = rows_v[e, pl.ds(16, 16)]
                    rows_v[e, pl.ds(0, 16)] = a * vv
                    rows_v[e, pl.ds(16, 16)] = b2 * vv

        def fire_scatter(b):
            eb, rows_v, sg, sw = bufs[b]
            for j in range(_SUB):
                pltpu.async_copy(rows_v.at[pl.ds(j * _CW, _CW)],
                                 acc.at[eb.at[3 * j]], sw, add=True)

        def wait_scatter(b):
            eb, rows_v, sg, sw = bufs[b]
            for j in range(_SUB):
                pltpu.make_async_copy(rows_v.at[pl.ds(j * _CW, _CW)],
                                      acc.at[eb.at[3 * j]], sw).wait()

        load(0, 0)
        fire_gather(0)

        def triple(i, carry):
            t0 = i * 3
            for k in range(3):
                t = t0 + k
                bn = (k + 1) % 3

                @pl.when(t >= 2)
                def _(bn=bn):
                    wait_scatter(bn)

                @pl.when(t < _CHUNKS - 1)
                def _(t=t, bn=bn):
                    load(t + 1, bn)
                    fire_gather(bn)

                wait_gather(k)
                scale(k)
                fire_scatter(k)
            return carry

        lax.fori_loop(0, _CHUNKS // 3, triple, 0)
        wait_scatter((_CHUNKS - 2) % 3)
        wait_scatter((_CHUNKS - 1) % 3)

    @pl.when(c == 0)
    def _():
        run(x_lo)

    @pl.when(c == 1)
    def _():
        run(x_hi)

    plsc.subcore_barrier()

    @pl.when(c == 0)
    def _():
        pltpu.sync_copy(acc.at[pl.ds(s * _ZROWS, _ZROWS)],
                        out_lo.at[pl.ds(s * _ZROWS, _ZROWS)])

    @pl.when(c == 1)
    def _():
        pltpu.sync_copy(acc.at[pl.ds(s * _ZROWS, _ZROWS)],
                        out_hi.at[pl.ds(s * _ZROWS, _ZROWS)])


_spmm = pl.kernel(
    _spmm_body,
    out_type=(jax.ShapeDtypeStruct((_NPAD, _H), jnp.float32),) * 2,
    mesh=_MESH,
    scratch_types=(
        [pltpu.VMEM((3 * _SUB, _CW), jnp.int32)] * 3
        + [pltpu.VMEM((_SUB * _CW, _H), jnp.float32)] * 3
        + [pltpu.VMEM_SHARED((_NPAD, _H), jnp.float32)]
        + [pltpu.SemaphoreType.DMA] * 6
    ),
    compiler_params=pltpu.CompilerParams(needs_layout_passes=False,
                                         use_tc_tiling_on_sc=False),
)


def _att_body(sd_lo, sd_hi, sm_lo, sm_hi, e_lo, e_hi, o_lo, o_hi):
    a, b = sd_lo[...], sd_hi[...]
    p, q = sm_lo[...], sm_hi[...]
    u1 = e_lo[...] + 1.0
    v1 = e_hi[...] + 1.0
    ms = (jnp.sum(a * u1, axis=1) + jnp.sum(b * v1, axis=1)) * (1.0 / _D)
    mm = (jnp.sum(p * u1, axis=1) + jnp.sum(q * v1, axis=1)) * (1.0 / _D)
    es = jnp.exp(ms)
    em = jnp.exp(mm)
    w = (es / (es + em))[:, None]
    o_lo[...] = w * a + (1.0 - w) * p
    o_hi[...] = w * b + (1.0 - w) * q


_BATT = 1024
_attention = pl.pallas_call(
    _att_body,
    grid=(_NPAD // _BATT,),
    in_specs=[pl.BlockSpec((_BATT, _H), lambda i: (i, 0))] * 6,
    out_specs=[pl.BlockSpec((_BATT, _H), lambda i: (i, 0))] * 2,
    out_shape=(jax.ShapeDtypeStruct((_NPAD, _H), jnp.float32),) * 2,
)


def _final_body(x0_lo, x0_hi, e1_lo, e1_hi, sd_lo, sd_hi, sm_lo, sm_hi,
                uu, ii, out,
                uidx, iidx,
                bx0ul, bx0uh, be1ul, be1uh, bsdul, bsduh, bsmul, bsmuh,
                bx0il, bx0ih, be1il, be1ih, bsdil, bsdih, bsmil, bsmih,
                stage, sv, sem):
    c = lax.axis_index("c")
    s = lax.axis_index("s")
    wid = s * _NC + c
    q0 = wid * _QPT
    pltpu.sync_copy(uu.at[pl.ds(q0, _QPT)], uidx)
    pltpu.sync_copy(ii.at[pl.ds(q0, _QPT)], iidx)
    tabs = (x0_lo, x0_hi, e1_lo, e1_hi, sd_lo, sd_hi, sm_lo, sm_hi)
    ubufs = (bx0ul, bx0uh, be1ul, be1uh, bsdul, bsduh, bsmul, bsmuh)
    ibufs = (bx0il, bx0ih, be1il, be1ih, bsdil, bsdih, bsmil, bsmih)
    handles = []
    for tab, bu, bi in zip(tabs, ubufs, ibufs):
        handles.append(pltpu.async_copy(tab.at[uidx], bu, sem))
        handles.append(pltpu.async_copy(tab.at[iidx], bi, sem))
    for h in handles:
        h.wait()

    def light(e, bufs):
        bx0l, bx0h, be1l, be1h, bsdl, bsdh, bsml, bsmh = bufs
        x0g = [bx0l[e, pl.ds(0, 16)], bx0l[e, pl.ds(16, 16)],
               bx0h[e, pl.ds(0, 16)], bx0h[e, pl.ds(16, 16)]]
        e1g = [be1l[e, pl.ds(0, 16)], be1l[e, pl.ds(16, 16)],
               be1h[e, pl.ds(0, 16)], be1h[e, pl.ds(16, 16)]]
        sdg = [bsdl[e, pl.ds(0, 16)], bsdl[e, pl.ds(16, 16)],
               bsdh[e, pl.ds(0, 16)], bsdh[e, pl.ds(16, 16)]]
        smg = [bsml[e, pl.ds(0, 16)], bsml[e, pl.ds(16, 16)],
               bsmh[e, pl.ds(0, 16)], bsmh[e, pl.ds(16, 16)]]
        ts = sdg[0] * (e1g[0] + 1.0)
        tm = smg[0] * (e1g[0] + 1.0)
        for g in range(1, 4):
            ts = ts + sdg[g] * (e1g[g] + 1.0)
            tm = tm + smg[g] * (e1g[g] + 1.0)
        ms = jnp.sum(ts) * (1.0 / _D)
        mm = jnp.sum(tm) * (1.0 / _D)
        es = jnp.exp(jnp.full((16,), ms))
        em = jnp.exp(jnp.full((16,), mm))
        w = es / (es + em)
        one_third = 1.0 / 3.0
        return [(x0g[g] + e1g[g] + w * sdg[g] + (1.0 - w) * smg[g]) * one_third
                for g in range(4)]

    def q(e, carry):
        lu = light(e, ubufs)
        li = light(e, ibufs)
        p = lu[0] * li[0]
        for g in range(1, 4):
            p = p + lu[g] * li[g]
        stage[pl.ds(e * 16, 16)] = p
        return carry

    lax.fori_loop(0, _QPT, q, 0)

    lanes = jnp.arange(16, dtype=jnp.int32) * 16
    for gq in range(_QPT // 16):
        base16 = lanes + (gq * 256)
        acc16 = plsc.load_gather(stage, [base16])
        for d in range(1, 16):
            acc16 = acc16 + plsc.load_gather(stage, [base16 + d])
        sv[pl.ds(gq * 16, 16)] = acc16
    pltpu.sync_copy(sv, out.at[pl.ds(q0, _QPT)])


_final = pl.kernel(
    _final_body,
    out_type=jax.ShapeDtypeStruct((_NQ,), jnp.float32),
    mesh=_MESH,
    scratch_types=[
        pltpu.VMEM((_QPT,), jnp.int32),
        pltpu.VMEM((_QPT,), jnp.int32),
    ] + [pltpu.VMEM((_QPT, _H), jnp.float32)] * 16 + [
        pltpu.VMEM((_QPT * 16,), jnp.float32),
        pltpu.VMEM((_QPT,), jnp.float32),
        pltpu.SemaphoreType.DMA,
    ],
    compiler_params=pltpu.CompilerParams(needs_layout_passes=False,
                                         use_tc_tiling_on_sc=False),
)


def _prep_edges(idx, val):
    pad = _EPAD - _E
    row = jnp.pad(idx[0], (0, pad)).reshape(_EROWS, _CW)
    col = jnp.pad(idx[1], (0, pad)).reshape(_EROWS, _CW)
    vbits = lax.bitcast_convert_type(jnp.pad(val, (0, pad)),
                                     jnp.int32).reshape(_EROWS, _CW)
    return jnp.stack([row, col, vbits], axis=1).reshape(3 * _EROWS, _CW)


def kernel(users, items, emb_user, emb_item, g_idx, g_val, s_idx, s_val):
    x0 = jnp.concatenate([emb_user, emb_item], axis=0)
    x0 = jnp.pad(x0, ((0, _NPAD - _N), (0, 0)))
    x_lo = x0[:, :_H]
    x_hi = x0[:, _H:]
    ged = _prep_edges(g_idx, g_val)
    sed = _prep_edges(s_idx, s_val)
    zblk = jnp.zeros((_ZROWS, _H), jnp.float32)

    sd_lo, sd_hi = _spmm(x_lo, x_hi, ged, zblk)
    sm_lo, sm_hi = _spmm(x_lo, x_hi, sed, zblk)
    e1_lo, e1_hi = _attention(sd_lo, sd_hi, sm_lo, sm_hi, x_lo, x_hi)
    sd2_lo, sd2_hi = _spmm(e1_lo, e1_hi, ged, zblk)
    sm2_lo, sm2_hi = _spmm(e1_lo, e1_hi, sed, zblk)

    scores = _final(x_lo, x_hi, e1_lo, e1_hi, sd2_lo, sd2_hi, sm2_lo, sm2_hi,
                    users, items + _N_USER)
    return scores


# speedup vs baseline: 4.2283x; 1.0487x over previous
import functools

import jax
import jax.numpy as jnp
from jax import lax
from jax.experimental import pallas as pl
from jax.experimental.pallas import tpu as pltpu
from jax.experimental.pallas import tpu_sc as plsc

_N_USER = 25000
_N = 50000
_D = 64
_H = 32
_E = 800000
_NC = 2
_NS = 16
_NPAD = 50176
_ZROWS = _NPAD // _NS
_SUB = 2
_CW = 128
_EPT = 52224
_EPAD = _EPT * _NS
_EROWS = _EPAD // _CW
_TROWS = _EPT // _CW
_CHUNKS = _TROWS // _SUB
_NQ = 4096
_QPT = _NQ // (_NC * _NS)

_MESH = plsc.VectorSubcoreMesh(core_axis_name="c", subcore_axis_name="s")


def _spmm_body(x_lo, x_hi, ed, zblk, out_lo, out_hi,
               eb0, eb1, eb2, eb3, eb4, eb5, rb0, rb1, rb2, acc,
               sl0, sl1, sl2, sg0, sg1, sg2, sw0, sw1, sw2):
    c = lax.axis_index("c")
    s = lax.axis_index("s")

    pltpu.sync_copy(zblk, acc.at[pl.ds(s * _ZROWS, _ZROWS)])
    plsc.subcore_barrier()

    ebs = (eb0, eb1, eb2, eb3, eb4, eb5)
    rbs = (rb0, rb1, rb2)
    sls = (sl0, sl1, sl2)
    sgs = (sg0, sg1, sg2)
    sws = (sw0, sw1, sw2)

    def run(x_tab):
        base0 = s * _TROWS

        def load_desc(t, ti):
            return pltpu.make_async_copy(
                ed.at[pl.ds((base0 + t * _SUB) * 3, 3 * _SUB)],
                ebs[ti % 6], sls[ti % 3])

        def gather_descs(t_dyn, ti):
            eb, rows_v, sg = ebs[ti % 6], rbs[ti % 3], sgs[ti % 3]
            del t_dyn
            return [pltpu.make_async_copy(x_tab.at[eb.at[3 * j + 1]],
                                          rows_v.at[pl.ds(j * _CW, _CW)], sg)
                    for j in range(_SUB)]

        def scatter_descs(ti):
            eb, rows_v, sw = ebs[ti % 6], rbs[ti % 3], sws[ti % 3]
            return [pltpu.make_async_copy(rows_v.at[pl.ds(j * _CW, _CW)],
                                          acc.at[eb.at[3 * j]], sw)
                    for j in range(_SUB)]

        def fire_scatter(ti):
            eb, rows_v, sw = ebs[ti % 6], rbs[ti % 3], sws[ti % 3]
            for j in range(_SUB):
                pltpu.async_copy(rows_v.at[pl.ds(j * _CW, _CW)],
                                 acc.at[eb.at[3 * j]], sw, add=True)

        def scale(ti):
            eb, rows_v = ebs[ti % 6], rbs[ti % 3]
            for j in range(_SUB):
                jv = jnp.full((16,), 3 * j + 2, jnp.int32)

                @functools.partial(plsc.parallel_loop, 0, _CW, unroll=8)
                def _(k, jv=jv, j=j):
                    kv = jnp.broadcast_to(k, (16,)).astype(jnp.int32)
                    vv = plsc.bitcast(plsc.load_gather(eb, [jv, kv]),
                                      jnp.float32)
                    e = j * _CW + k
                    a = rows_v[e, pl.ds(0, 16)]
                    b2 = rows_v[e, pl.ds(16, 16)]
                    rows_v[e, pl.ds(0, 16)] = a * vv
                    rows_v[e, pl.ds(16, 16)] = b2 * vv

        load_desc(0, 0).start()
        load_desc(1, 1).start()
        load_desc(0, 0).wait()
        for h in gather_descs(0, 0):
            h.start()

        def six(i, carry):
            t0 = i * 6
            for k in range(6):
                t = t0 + k

                @pl.when(t >= 2)
                def _(k=k):
                    for h in scatter_descs(k + 4):
                        h.wait()

                @pl.when(t < _CHUNKS - 2)
                def _(t=t, k=k):
                    load_desc(t + 2, k + 2).start()

                @pl.when(t < _CHUNKS - 1)
                def _(t=t, k=k):
                    load_desc(t + 1, k + 1).wait()
                    for h in gather_descs(t + 1, k + 1):
                        h.start()

                for h in gather_descs(t, k):
                    h.wait()
                scale(k)
                fire_scatter(k)
            return carry

        lax.fori_loop(0, _CHUNKS // 6, six, 0)
        for h in scatter_descs(_CHUNKS - 2):
            h.wait()
        for h in scatter_descs(_CHUNKS - 1):
            h.wait()

    @pl.when(c == 0)
    def _():
        run(x_lo)

    @pl.when(c == 1)
    def _():
        run(x_hi)

    plsc.subcore_barrier()

    @pl.when(c == 0)
    def _():
        pltpu.sync_copy(acc.at[pl.ds(s * _ZROWS, _ZROWS)],
                        out_lo.at[pl.ds(s * _ZROWS, _ZROWS)])

    @pl.when(c == 1)
    def _():
        pltpu.sync_copy(acc.at[pl.ds(s * _ZROWS, _ZROWS)],
                        out_hi.at[pl.ds(s * _ZROWS, _ZROWS)])


_spmm = pl.kernel(
    _spmm_body,
    out_type=(jax.ShapeDtypeStruct((_NPAD, _H), jnp.float32),) * 2,
    mesh=_MESH,
    scratch_types=(
        [pltpu.VMEM((3 * _SUB, _CW), jnp.int32)] * 6
        + [pltpu.VMEM((_SUB * _CW, _H), jnp.float32)] * 3
        + [pltpu.VMEM_SHARED((_NPAD, _H), jnp.float32)]
        + [pltpu.SemaphoreType.DMA] * 9
    ),
    compiler_params=pltpu.CompilerParams(needs_layout_passes=False,
                                         use_tc_tiling_on_sc=False),
)


def _att_body(sd_lo, sd_hi, sm_lo, sm_hi, e_lo, e_hi, o_lo, o_hi):
    a, b = sd_lo[...], sd_hi[...]
    p, q = sm_lo[...], sm_hi[...]
    u1 = e_lo[...] + 1.0
    v1 = e_hi[...] + 1.0
    ms = (jnp.sum(a * u1, axis=1) + jnp.sum(b * v1, axis=1)) * (1.0 / _D)
    mm = (jnp.sum(p * u1, axis=1) + jnp.sum(q * v1, axis=1)) * (1.0 / _D)
    es = jnp.exp(ms)
    em = jnp.exp(mm)
    w = (es / (es + em))[:, None]
    o_lo[...] = w * a + (1.0 - w) * p
    o_hi[...] = w * b + (1.0 - w) * q


_BATT = 1024
_attention = pl.pallas_call(
    _att_body,
    grid=(_NPAD // _BATT,),
    in_specs=[pl.BlockSpec((_BATT, _H), lambda i: (i, 0))] * 6,
    out_specs=[pl.BlockSpec((_BATT, _H), lambda i: (i, 0))] * 2,
    out_shape=(jax.ShapeDtypeStruct((_NPAD, _H), jnp.float32),) * 2,
)


def _final_body(x0_lo, x0_hi, e1_lo, e1_hi, sd_lo, sd_hi, sm_lo, sm_hi,
                uu, ii, out,
                uidx, iidx,
                bx0ul, bx0uh, be1ul, be1uh, bsdul, bsduh, bsmul, bsmuh,
                bx0il, bx0ih, be1il, be1ih, bsdil, bsdih, bsmil, bsmih,
                stage, sv, sem):
    c = lax.axis_index("c")
    s = lax.axis_index("s")
    wid = s * _NC + c
    q0 = wid * _QPT
    pltpu.sync_copy(uu.at[pl.ds(q0, _QPT)], uidx)
    pltpu.sync_copy(ii.at[pl.ds(q0, _QPT)], iidx)
    tabs = (x0_lo, x0_hi, e1_lo, e1_hi, sd_lo, sd_hi, sm_lo, sm_hi)
    ubufs = (bx0ul, bx0uh, be1ul, be1uh, bsdul, bsduh, bsmul, bsmuh)
    ibufs = (bx0il, bx0ih, be1il, be1ih, bsdil, bsdih, bsmil, bsmih)
    handles = []
    for tab, bu, bi in zip(tabs, ubufs, ibufs):
        handles.append(pltpu.async_copy(tab.at[uidx], bu, sem))
        handles.append(pltpu.async_copy(tab.at[iidx], bi, sem))
    for h in handles:
        h.wait()

    def light(e, bufs):
        bx0l, bx0h, be1l, be1h, bsdl, bsdh, bsml, bsmh = bufs
        x0g = [bx0l[e, pl.ds(0, 16)], bx0l[e, pl.ds(16, 16)],
               bx0h[e, pl.ds(0, 16)], bx0h[e, pl.ds(16, 16)]]
        e1g = [be1l[e, pl.ds(0, 16)], be1l[e, pl.ds(16, 16)],
               be1h[e, pl.ds(0, 16)], be1h[e, pl.ds(16, 16)]]
        sdg = [bsdl[e, pl.ds(0, 16)], bsdl[e, pl.ds(16, 16)],
               bsdh[e, pl.ds(0, 16)], bsdh[e, pl.ds(16, 16)]]
        smg = [bsml[e, pl.ds(0, 16)], bsml[e, pl.ds(16, 16)],
               bsmh[e, pl.ds(0, 16)], bsmh[e, pl.ds(16, 16)]]
        ts = sdg[0] * (e1g[0] + 1.0)
        tm = smg[0] * (e1g[0] + 1.0)
        for g in range(1, 4):
            ts = ts + sdg[g] * (e1g[g] + 1.0)
            tm = tm + smg[g] * (e1g[g] + 1.0)
        ms = jnp.sum(ts) * (1.0 / _D)
        mm = jnp.sum(tm) * (1.0 / _D)
        es = jnp.exp(jnp.full((16,), ms))
        em = jnp.exp(jnp.full((16,), mm))
        w = es / (es + em)
        one_third = 1.0 / 3.0
        return [(x0g[g] + e1g[g] + w * sdg[g] + (1.0 - w) * smg[g]) * one_third
                for g in range(4)]

    def q(e, carry):
        lu = light(e, ubufs)
        li = light(e, ibufs)
        p = lu[0] * li[0]
        for g in range(1, 4):
            p = p + lu[g] * li[g]
        stage[pl.ds(e * 16, 16)] = p
        return carry

    lax.fori_loop(0, _QPT, q, 0)

    lanes = jnp.arange(16, dtype=jnp.int32) * 16
    for gq in range(_QPT // 16):
        base16 = lanes + (gq * 256)
        acc16 = plsc.load_gather(stage, [base16])
        for d in range(1, 16):
            acc16 = acc16 + plsc.load_gather(stage, [base16 + d])
        sv[pl.ds(gq * 16, 16)] = acc16
    pltpu.sync_copy(sv, out.at[pl.ds(q0, _QPT)])


_final = pl.kernel(
    _final_body,
    out_type=jax.ShapeDtypeStruct((_NQ,), jnp.float32),
    mesh=_MESH,
    scratch_types=[
        pltpu.VMEM((_QPT,), jnp.int32),
        pltpu.VMEM((_QPT,), jnp.int32),
    ] + [pltpu.VMEM((_QPT, _H), jnp.float32)] * 16 + [
        pltpu.VMEM((_QPT * 16,), jnp.float32),
        pltpu.VMEM((_QPT,), jnp.float32),
        pltpu.SemaphoreType.DMA,
    ],
    compiler_params=pltpu.CompilerParams(needs_layout_passes=False,
                                         use_tc_tiling_on_sc=False),
)


def _prep_edges(idx, val):
    pad = _EPAD - _E
    row = jnp.pad(idx[0], (0, pad)).reshape(_EROWS, _CW)
    col = jnp.pad(idx[1], (0, pad)).reshape(_EROWS, _CW)
    vbits = lax.bitcast_convert_type(jnp.pad(val, (0, pad)),
                                     jnp.int32).reshape(_EROWS, _CW)
    return jnp.stack([row, col, vbits], axis=1).reshape(3 * _EROWS, _CW)


def kernel(users, items, emb_user, emb_item, g_idx, g_val, s_idx, s_val):
    x0 = jnp.concatenate([emb_user, emb_item], axis=0)
    x0 = jnp.pad(x0, ((0, _NPAD - _N), (0, 0)))
    x_lo = x0[:, :_H]
    x_hi = x0[:, _H:]
    ged = _prep_edges(g_idx, g_val)
    sed = _prep_edges(s_idx, s_val)
    zblk = jnp.zeros((_ZROWS, _H), jnp.float32)

    sd_lo, sd_hi = _spmm(x_lo, x_hi, ged, zblk)
    sm_lo, sm_hi = _spmm(x_lo, x_hi, sed, zblk)
    e1_lo, e1_hi = _attention(sd_lo, sd_hi, sm_lo, sm_hi, x_lo, x_hi)
    sd2_lo, sd2_hi = _spmm(e1_lo, e1_hi, ged, zblk)
    sm2_lo, sm2_hi = _spmm(e1_lo, e1_hi, sed, zblk)

    scores = _final(x_lo, x_hi, e1_lo, e1_hi, sd2_lo, sd2_hi, sm2_lo, sm2_hi,
                    users, items + _N_USER)
    return scores
